# Initial kernel scaffold; baseline (speedup 1.0000x reference)
import jax
import jax.numpy as jnp
from jax.experimental import pallas as pl


def kernel(x, table):
    raise NotImplementedError("write your pallas kernel here")


# speedup vs baseline: 1.6946x; 1.6946x over previous
import jax
import jax.numpy as jnp
from jax.experimental import pallas as pl
from jax.experimental.pallas import tpu as pltpu

_BS = 512
_POS_OFFSET = 2


def _tc_body(table_ref, x_ref, o_ref, tbl_v, sems):
    j = pl.program_id(0)
    n = pl.num_programs(0)

    def _copy(k, slot):
        return pltpu.make_async_copy(
            table_ref.at[pl.ds(k * _BS, _BS + 8), :],
            tbl_v.at[slot],
            sems.at[slot],
        )

    @pl.when(j == 0)
    def _():
        _copy(0, 0).start()

    @pl.when(j + 1 < n)
    def _():
        _copy(j + 1, jax.lax.rem(j + 1, 2)).start()

    slot = jax.lax.rem(j, 2)
    _copy(j, slot).wait()
    o_ref[...] = x_ref[...] + tbl_v[slot, _POS_OFFSET:_POS_OFFSET + _BS, :][None, :, :]


@jax.jit
def kernel(x, table):
    B, S, D = x.shape
    n = S // _BS
    return pl.pallas_call(
        _tc_body,
        grid=(n,),
        in_specs=[
            pl.BlockSpec(memory_space=pl.ANY),
            pl.BlockSpec((B, _BS, D), lambda j: (0, j, 0)),
        ],
        out_specs=pl.BlockSpec((B, _BS, D), lambda j: (0, j, 0)),
        out_shape=jax.ShapeDtypeStruct(x.shape, x.dtype),
        scratch_shapes=[
            pltpu.VMEM((2, _BS + 8, D), x.dtype),
            pltpu.SemaphoreType.DMA((2,)),
        ],
    )(table, x)
